# baseline (speedup 1.0000x reference)
import functools
import jax
import jax.numpy as jnp
from jax import lax
from jax.experimental import pallas as pl
from jax.experimental.pallas import tpu as pltpu
from jax.experimental.pallas import tpu_sc as plsc

_B = 16384
_C = 26
_D = 64
_NC = 2
_NS = 16
_NW = _NC * _NS
_BPW = _B // _NW
_CHUNK = 128
_NCHUNK = _BPW // _CHUNK
_LANES = 16


def _gather_kernel(table_hbm, label_hbm, out_hbm, lab_v, idx_v, rows_v, sem):
    wid = lax.axis_index("s") * _NC + lax.axis_index("c")
    base = wid * _BPW

    pltpu.sync_copy(label_hbm.at[pl.ds(base, _BPW)], lab_v)

    lane = lax.iota(jnp.int32, _LANES)
    for j in range(_BPW // _LANES):
        lab = lab_v[pl.ds(j * _LANES, _LANES)]
        gidx = (base + j * _LANES + lane) * _C + lab
        r, col = divmod(j * _LANES, _CHUNK)
        idx_v[r, pl.ds(col, _LANES)] = gidx

    copies = []
    for r in range(_NCHUNK):
        copies.append(
            pltpu.async_copy(
                table_hbm.at[idx_v.at[r]],
                rows_v.at[pl.ds(r * _CHUNK, _CHUNK)],
                sem,
            )
        )
    for c in copies:
        c.wait()

    pltpu.sync_copy(rows_v, out_hbm.at[pl.ds(base, _BPW)])


@jax.jit
def _maskout(table, label):
    mesh = plsc.VectorSubcoreMesh(core_axis_name="c", subcore_axis_name="s")
    return pl.kernel(
        _gather_kernel,
        mesh=mesh,
        out_type=jax.ShapeDtypeStruct((_B, _D), jnp.float32),
        scratch_types=[
            pltpu.VMEM((_BPW,), jnp.int32),
            pltpu.VMEM((_NCHUNK, _CHUNK), jnp.int32),
            pltpu.VMEM((_BPW, _D), jnp.float32),
            pltpu.SemaphoreType.DMA,
        ],
        compiler_params=pltpu.CompilerParams(use_tc_tiling_on_sc=False),
    )(table, label)


def kernel(x, label):
    table = x.reshape(_B * _C, _D)
    return _maskout(table, label.astype(jnp.int32))


# speedup vs baseline: 1.1825x; 1.1825x over previous
import functools
import jax
import jax.numpy as jnp
from jax import lax
from jax.experimental import pallas as pl
from jax.experimental.pallas import tpu as pltpu
from jax.experimental.pallas import tpu_sc as plsc

_B = 16384
_C = 26
_D = 64
_NC = 2
_NS = 16
_NW = _NC * _NS
_BPW = _B // _NW
_LANES = 16


def _gather_kernel(x_hbm, label_hbm, out_hbm, lab_v, rows_v, sem):
    wid = lax.axis_index("s") * _NC + lax.axis_index("c")
    base = wid * _BPW

    pltpu.sync_copy(label_hbm.at[pl.ds(base, _BPW)], lab_v)

    def body(i, _):
        r0 = i * _LANES
        labs = lab_v[pl.ds(r0, _LANES)]
        for u in range(_LANES):
            c = labs[u]
            pltpu.async_copy(x_hbm.at[base + r0 + u, c], rows_v.at[r0 + u], sem)
        return ()

    lax.fori_loop(0, _BPW // _LANES, body, (), unroll=False)

    pltpu.make_async_copy(out_hbm.at[pl.ds(base, _BPW)], rows_v, sem).wait()

    pltpu.sync_copy(rows_v, out_hbm.at[pl.ds(base, _BPW)])


@jax.jit
def _maskout(x, label):
    mesh = plsc.VectorSubcoreMesh(core_axis_name="c", subcore_axis_name="s")
    return pl.kernel(
        _gather_kernel,
        mesh=mesh,
        out_type=jax.ShapeDtypeStruct((_B, _D), jnp.float32),
        scratch_types=[
            pltpu.VMEM((_BPW,), jnp.int32),
            pltpu.VMEM((_BPW, _D), jnp.float32),
            pltpu.SemaphoreType.DMA,
        ],
        compiler_params=pltpu.CompilerParams(use_tc_tiling_on_sc=True),
    )(x, label)


def kernel(x, label):
    return _maskout(x, label.astype(jnp.int32))
